# Initial kernel scaffold; baseline (speedup 1.0000x reference)
import functools

import jax
import jax.numpy as jnp
from jax import lax
from jax.experimental import pallas as pl
from jax.experimental.pallas import tpu as pltpu
from jax.experimental.pallas import tpu_sc as plsc

_NUM_USER = 25000
_NUM_ITEM = 25000
_N = _NUM_USER + _NUM_ITEM
_D = 64
_E = 800000
_LAYERS = 3

_HALF = 25000
_PAD = 8
_HP = _HALF + _PAD
_NP = 2 * _HP

_NTILES = 16
_CHUNK = 128
_EPAD = 800768
_EPW = _EPAD // _NTILES
_NCHUNK = _EPW // _CHUNK

_ZROWS = 521
_TROWS = _HP // _NTILES


def _mesh():
    return plsc.VectorSubcoreMesh(core_axis_name="c", subcore_axis_name="s")


@functools.partial(
    pl.kernel,
    mesh=_mesh(),
    out_type=jax.ShapeDtypeStruct((_NP, _D), jnp.float32),
    scratch_types=[
        pltpu.VMEM_SHARED((_HP, _D), jnp.float32),
        pltpu.VMEM((_CHUNK,), jnp.int32),
        pltpu.VMEM((_CHUNK,), jnp.int32),
        pltpu.VMEM((_CHUNK,), jnp.float32),
        pltpu.VMEM((_CHUNK, _D), jnp.float32),
        pltpu.VMEM((_ZROWS, _D), jnp.float32),
        pltpu.SemaphoreType.DMA,
    ],
)
def _layer(src_hbm, dst_hbm, w_hbm, emb_hbm, out_hbm,
           acc, srcv, dstv, wv, rows, zv, sem):
    c = lax.axis_index("c")
    s = lax.axis_index("s")
    cbase = c * _HALF

    def _zero_body(i, _):
        r = i // 4
        j = i % 4
        zv[r, pl.ds(j * 16, 16)] = jnp.zeros((16,), jnp.float32)
        return 0

    lax.fori_loop(0, _ZROWS * 4, _zero_body, 0, unroll=4)
    for k in range(3):
        pltpu.sync_copy(zv, acc.at[pl.ds(s * _TROWS + k * _ZROWS, _ZROWS)])
    plsc.subcore_barrier()

    ebase = s * _EPW

    def _chunk_body(ci, _):
        eoff = ebase + ci * _CHUNK
        pltpu.sync_copy(src_hbm.at[pl.ds(eoff, _CHUNK)], srcv)
        pltpu.sync_copy(dst_hbm.at[pl.ds(eoff, _CHUNK)], dstv)
        pltpu.sync_copy(w_hbm.at[pl.ds(eoff, _CHUNK)], wv)
        for g in range(_CHUNK // 16):
            sl = pl.ds(g * 16, 16)
            sv = srcv[sl]
            srcv[sl] = jnp.where(sv >= _HALF, sv + _PAD, sv)
            dv = dstv[sl]
            inr = (dv >= cbase) & (dv < cbase + _HALF)
            dstv[sl] = jnp.where(inr, dv - cbase, 0)
            wv[sl] = jnp.where(inr, wv[sl], jnp.zeros((16,), jnp.float32))
        pltpu.async_copy(emb_hbm.at[srcv], rows, sem).wait()

        def _scale_body(r, _):
            wr = wv[r]
            for j in range(4):
                sl2 = pl.ds(j * 16, 16)
                rows[r, sl2] = rows[r, sl2] * wr
            return 0

        lax.fori_loop(0, _CHUNK, _scale_body, 0, unroll=4)
        pltpu.sync_copy(rows, acc.at[dstv], add=True)
        return 0

    lax.fori_loop(0, _NCHUNK, _chunk_body, 0)
    plsc.subcore_barrier()

    for k in range(3):
        off = s * _TROWS + k * _ZROWS
        pltpu.sync_copy(acc.at[pl.ds(off, _ZROWS)],
                        out_hbm.at[pl.ds(c * _HP + off, _ZROWS)])


def kernel(edge_index, edge_weight, user_table, item_table):
    src = edge_index[1].astype(jnp.int32)
    dst = edge_index[0].astype(jnp.int32)
    w = edge_weight.astype(jnp.float32)
    pad = _EPAD - _E
    src = jnp.concatenate([src, jnp.zeros((pad,), jnp.int32)])
    dst = jnp.concatenate([dst, jnp.zeros((pad,), jnp.int32)])
    w = jnp.concatenate([w, jnp.zeros((pad,), jnp.float32)])

    z8 = jnp.zeros((_PAD, _D), jnp.float32)
    emb = jnp.concatenate([user_table, z8, item_table, z8], axis=0)

    total = emb
    cur = emb
    for _ in range(_LAYERS):
        cur = _layer(src, dst, w, cur)
        total = total + cur
    mean = total * (1.0 / (_LAYERS + 1))
    return (mean[:_NUM_USER], mean[_HP:_HP + _NUM_ITEM])


# speedup vs baseline: 2.5330x; 2.5330x over previous
import functools

import jax
import jax.numpy as jnp
from jax import lax
from jax.experimental import pallas as pl
from jax.experimental.pallas import tpu as pltpu
from jax.experimental.pallas import tpu_sc as plsc

_NUM_USER = 25000
_NUM_ITEM = 25000
_N = _NUM_USER + _NUM_ITEM
_D = 64
_E = 800000
_LAYERS = 3

_HALF = 25000
_PAD = 88
_HP = _HALF + _PAD
_NP = 2 * _HP

_NTILES = 16
_CHUNK = 128
_EPAD = 800768
_EPW = _EPAD // _NTILES
_NCHUNK = _EPW // _CHUNK

_ZROWS = 112
_TROWS = _HP // _NTILES


def _mesh():
    return plsc.VectorSubcoreMesh(core_axis_name="c", subcore_axis_name="s")


@functools.partial(
    pl.kernel,
    mesh=_mesh(),
    out_type=jax.ShapeDtypeStruct((_NP, _D), jnp.float32),
    scratch_types=[
        pltpu.VMEM_SHARED((_HP, _D), jnp.float32),
        pltpu.VMEM((_CHUNK,), jnp.int32),
        pltpu.VMEM((_CHUNK,), jnp.int32),
        pltpu.VMEM((_CHUNK,), jnp.float32),
        pltpu.VMEM((_CHUNK, _D), jnp.float32),
        pltpu.VMEM((_ZROWS, _D), jnp.float32),
        pltpu.SemaphoreType.DMA,
    ],
    compiler_params=pltpu.CompilerParams(use_tc_tiling_on_sc=False),
)
def _layer(src_hbm, dst_hbm, w_hbm, emb_hbm, out_hbm,
           acc, srcv, dstv, wv, rows, zv, sem):
    c = lax.axis_index("c")
    s = lax.axis_index("s")
    cbase = c * _HALF

    def _zero_body(i, _):
        r = i // 4
        j = i % 4
        zv[r, pl.ds(j * 16, 16)] = jnp.zeros((16,), jnp.float32)
        return 0

    lax.fori_loop(0, _ZROWS * 4, _zero_body, 0, unroll=4)

    def _zinit(k, _):
        pltpu.sync_copy(zv, acc.at[pl.ds(s * _TROWS + k * _ZROWS, _ZROWS)])
        return 0

    lax.fori_loop(0, _TROWS // _ZROWS, _zinit, 0)
    plsc.subcore_barrier()

    ebase = s * _EPW

    def _chunk_body(ci, _):
        eoff = ebase + ci * _CHUNK
        pltpu.sync_copy(src_hbm.at[pl.ds(eoff, _CHUNK)], srcv)
        pltpu.sync_copy(dst_hbm.at[pl.ds(eoff, _CHUNK)], dstv)
        pltpu.sync_copy(w_hbm.at[pl.ds(eoff, _CHUNK)], wv)
        for g in range(_CHUNK // 16):
            sl = pl.ds(g * 16, 16)
            sv = srcv[sl]
            srcv[sl] = jnp.where(sv >= _HALF, sv + _PAD, sv)
            dv = dstv[sl]
            inr = (dv >= cbase) & (dv < cbase + _HALF)
            dstv[sl] = jnp.where(inr, dv - cbase, 0)
            wv[sl] = jnp.where(inr, wv[sl], jnp.zeros((16,), jnp.float32))
        pltpu.async_copy(emb_hbm.at[srcv], rows, sem).wait()

        def _scale_body(g, _):
            wg = wv[pl.ds(g * 16, 16)]
            rbase = g * 16
            for i in range(16):
                wr = wg[i]
                for j in range(4):
                    sl2 = pl.ds(j * 16, 16)
                    rows[rbase + i, sl2] = rows[rbase + i, sl2] * wr
            return 0

        lax.fori_loop(0, _CHUNK // 16, _scale_body, 0)
        pltpu.sync_copy(rows, acc.at[dstv], add=True)
        return 0

    lax.fori_loop(0, _NCHUNK, _chunk_body, 0)
    plsc.subcore_barrier()

    def _wback(k, _):
        off = s * _TROWS + k * _ZROWS
        pltpu.sync_copy(acc.at[pl.ds(off, _ZROWS)],
                        out_hbm.at[pl.ds(c * _HP + off, _ZROWS)])
        return 0

    lax.fori_loop(0, _TROWS // _ZROWS, _wback, 0)


def kernel(edge_index, edge_weight, user_table, item_table):
    src = edge_index[1].astype(jnp.int32)
    dst = edge_index[0].astype(jnp.int32)
    w = edge_weight.astype(jnp.float32)
    pad = _EPAD - _E
    src = jnp.concatenate([src, jnp.zeros((pad,), jnp.int32)])
    dst = jnp.concatenate([dst, jnp.zeros((pad,), jnp.int32)])
    w = jnp.concatenate([w, jnp.zeros((pad,), jnp.float32)])

    z8 = jnp.zeros((_PAD, _D), jnp.float32)
    emb = jnp.concatenate([user_table, z8, item_table, z8], axis=0)

    total = emb
    cur = emb
    for _ in range(_LAYERS):
        cur = _layer(src, dst, w, cur)
        total = total + cur
    mean = total * (1.0 / (_LAYERS + 1))
    return (mean[:_NUM_USER], mean[_HP:_HP + _NUM_ITEM])


# speedup vs baseline: 2.7358x; 1.0801x over previous
import functools

import jax
import jax.numpy as jnp
from jax import lax
from jax.experimental import pallas as pl
from jax.experimental.pallas import tpu as pltpu
from jax.experimental.pallas import tpu_sc as plsc

_NUM_USER = 25000
_NUM_ITEM = 25000
_D = 64
_E = 800000
_LAYERS = 3

_HALF = 25000
_PAD = 88
_HP = _HALF + _PAD
_NP = 2 * _HP

_NTILES = 16
_CHUNK = 128
_EPAD = 802816
_EPW = _EPAD // _NTILES
_NCHUNK = _EPW // _CHUNK

_ZROWS = 112
_TROWS = _HP // _NTILES


def _mesh():
    return plsc.VectorSubcoreMesh(core_axis_name="c", subcore_axis_name="s")


@functools.partial(
    pl.kernel,
    mesh=_mesh(),
    out_type=jax.ShapeDtypeStruct((_NP, _D), jnp.float32),
    scratch_types=[
        pltpu.VMEM_SHARED((_HP, _D), jnp.float32),
        pltpu.VMEM((3 * _CHUNK,), jnp.int32),
        pltpu.VMEM((3 * _CHUNK,), jnp.int32),
        pltpu.VMEM((_CHUNK,), jnp.int32),
        pltpu.VMEM((_CHUNK,), jnp.int32),
        pltpu.VMEM((_CHUNK,), jnp.int32),
        pltpu.VMEM((_CHUNK,), jnp.int32),
        pltpu.VMEM((_CHUNK,), jnp.float32),
        pltpu.VMEM((_CHUNK,), jnp.float32),
        pltpu.VMEM((_CHUNK, _D), jnp.float32),
        pltpu.VMEM((_CHUNK, _D), jnp.float32),
        pltpu.VMEM((_ZROWS, _D), jnp.float32),
        pltpu.SemaphoreType.DMA,
        pltpu.SemaphoreType.DMA,
    ],
    compiler_params=pltpu.CompilerParams(use_tc_tiling_on_sc=False,
                                         needs_layout_passes=False),
)
def _layer(e_hbm, emb_hbm, out_hbm,
           acc, eb0, eb1, sv0, sv1, dv0, dv1, wv0, wv1,
           rows0, rows1, zv, sem0, sem1):
    c = lax.axis_index("c")
    s = lax.axis_index("s")
    cbase = c * _HALF

    def _zero_body(i, _):
        r = i // 4
        j = i % 4
        zv[r, pl.ds(j * 16, 16)] = jnp.zeros((16,), jnp.float32)
        return 0

    lax.fori_loop(0, _ZROWS * 4, _zero_body, 0, unroll=4)

    def _zinit(k, _):
        pltpu.sync_copy(zv, acc.at[pl.ds(s * _TROWS + k * _ZROWS, _ZROWS)])
        return 0

    lax.fori_loop(0, _TROWS // _ZROWS, _zinit, 0)
    plsc.subcore_barrier()

    cbchunk = s * _NCHUNK

    def _issue(ci, eb, sv, dv, wv, rows, sem):
        pltpu.sync_copy(e_hbm.at[pl.ds((cbchunk + ci) * 3 * _CHUNK,
                                       3 * _CHUNK)], eb)
        for g in range(_CHUNK // 16):
            svg = eb[pl.ds(g * 16, 16)]
            sv[pl.ds(g * 16, 16)] = jnp.where(svg >= _HALF, svg + _PAD, svg)
            dvg = eb[pl.ds(_CHUNK + g * 16, 16)]
            wg = plsc.bitcast(eb[pl.ds(2 * _CHUNK + g * 16, 16)], jnp.float32)
            inr = (dvg >= cbase) & (dvg < cbase + _HALF)
            spread = _HALF + (g % 4) * 16 + lax.iota(jnp.int32, 16)
            dv[pl.ds(g * 16, 16)] = jnp.where(inr, dvg - cbase, spread)
            wv[pl.ds(g * 16, 16)] = jnp.where(
                inr, wg, jnp.zeros((16,), jnp.float32))
        pltpu.async_copy(emb_hbm.at[sv], rows, sem)

    def _finish(sv, dv, wv, rows, sem):
        pltpu.make_async_copy(emb_hbm.at[sv], rows, sem).wait()

        def _scale_body(g, _):
            wg = wv[pl.ds(g * 16, 16)]
            rbase = g * 16
            for i in range(16):
                wr = wg[i]
                for j in range(4):
                    sl2 = pl.ds(j * 16, 16)
                    rows[rbase + i, sl2] = rows[rbase + i, sl2] * wr
            return 0

        lax.fori_loop(0, _CHUNK // 16, _scale_body, 0)
        pltpu.sync_copy(rows, acc.at[dv], add=True)

    _issue(0, eb0, sv0, dv0, wv0, rows0, sem0)

    def _pipe_body(k, _):
        _issue(2 * k + 1, eb1, sv1, dv1, wv1, rows1, sem1)
        _finish(sv0, dv0, wv0, rows0, sem0)

        @pl.when(k < _NCHUNK // 2 - 1)
        def _():
            _issue(2 * k + 2, eb0, sv0, dv0, wv0, rows0, sem0)

        _finish(sv1, dv1, wv1, rows1, sem1)
        return 0

    lax.fori_loop(0, _NCHUNK // 2, _pipe_body, 0)
    plsc.subcore_barrier()

    def _wback(k, _):
        off = s * _TROWS + k * _ZROWS
        pltpu.sync_copy(acc.at[pl.ds(off, _ZROWS)],
                        out_hbm.at[pl.ds(c * _HP + off, _ZROWS)])
        return 0

    lax.fori_loop(0, _TROWS // _ZROWS, _wback, 0)


def kernel(edge_index, edge_weight, user_table, item_table):
    src = edge_index[1].astype(jnp.int32)
    dst = edge_index[0].astype(jnp.int32)
    w32 = lax.bitcast_convert_type(edge_weight.astype(jnp.float32), jnp.int32)
    pad = _EPAD - _E
    src = jnp.concatenate([src, jnp.zeros((pad,), jnp.int32)])
    dst = jnp.concatenate([dst, jnp.zeros((pad,), jnp.int32)])
    w32 = jnp.concatenate([w32, jnp.zeros((pad,), jnp.int32)])
    e = jnp.stack([src.reshape(-1, _CHUNK), dst.reshape(-1, _CHUNK),
                   w32.reshape(-1, _CHUNK)], axis=1).reshape(-1)

    z8 = jnp.zeros((_PAD, _D), jnp.float32)
    emb = jnp.concatenate([user_table, z8, item_table, z8], axis=0)

    total = emb
    cur = emb
    for _ in range(_LAYERS):
        cur = _layer(e, cur)
        total = total + cur
    mean = total * (1.0 / (_LAYERS + 1))
    return (mean[:_NUM_USER], mean[_HP:_HP + _NUM_ITEM])


# speedup vs baseline: 4.4444x; 1.6246x over previous
import functools

import jax
import jax.numpy as jnp
from jax import lax
from jax.experimental import pallas as pl
from jax.experimental.pallas import tpu as pltpu
from jax.experimental.pallas import tpu_sc as plsc

_NUM_USER = 25000
_NUM_ITEM = 25000
_D = 64
_E = 800000
_LAYERS = 3

_HALF = 25000
_PAD = 88
_HP = _HALF + _PAD
_NP = 2 * _HP

_NTILES = 16
_CHUNK = 128
_EPAD = 802816
_EPW = _EPAD // _NTILES
_NCHUNK = _EPW // _CHUNK
_NCHT = _EPAD // _CHUNK
_REGW = (_NCHT + 16) * 384

_ZROWS = 112
_TROWS = _HP // _NTILES


def _mesh():
    return plsc.VectorSubcoreMesh(core_axis_name="c", subcore_axis_name="s")


_CPARAMS = pltpu.CompilerParams(use_tc_tiling_on_sc=False,
                                needs_layout_passes=False)


@functools.partial(
    pl.kernel,
    mesh=_mesh(),
    out_type=(jax.ShapeDtypeStruct((2 * _REGW,), jnp.int32),
              jax.ShapeDtypeStruct((32,), jnp.int32)),
    scratch_types=[
        pltpu.VMEM((3 * _CHUNK,), jnp.int32),
        pltpu.VMEM((3 * _CHUNK,), jnp.int32),
        pltpu.VMEM((144,), jnp.int32),
        pltpu.VMEM((144,), jnp.int32),
        pltpu.VMEM((144,), jnp.int32),
        pltpu.VMEM((16,), jnp.int32),
        pltpu.SMEM((8,), jnp.int32),
        pltpu.SemaphoreType.DMA,
        pltpu.SemaphoreType.DMA,
    ],
    compiler_params=_CPARAMS,
)
def _partition(e_hbm, part_hbm, counts_hbm, eb0, eb1, st_s, st_d, st_w,
               cbuf, cnt, sem0, sem1):
    c = lax.axis_index("c")
    s = lax.axis_index("s")
    cbase = c * _HALF
    obase = c * _REGW

    @pl.when(s == 0)
    def _():
        cnt[0] = 0

    plsc.subcore_barrier()

    iota = lax.iota(jnp.int32, 16)

    def _flush():
        slot = plsc.fetch_and_add(cnt.at[0], 1, subcore_id=0)
        off = obase + slot * 384
        pltpu.sync_copy(st_s.at[pl.ds(0, _CHUNK)],
                        part_hbm.at[pl.ds(off, _CHUNK)])
        pltpu.sync_copy(st_d.at[pl.ds(0, _CHUNK)],
                        part_hbm.at[pl.ds(off + _CHUNK, _CHUNK)])
        pltpu.sync_copy(st_w.at[pl.ds(0, _CHUNK)],
                        part_hbm.at[pl.ds(off + 2 * _CHUNK, _CHUNK)])

    def _eload(mc, eb, sem):
        pltpu.async_copy(e_hbm.at[pl.ds((s * _NCHUNK + mc) * 384, 384)],
                         eb, sem)

    def _ewait(mc, eb, sem):
        pltpu.make_async_copy(
            e_hbm.at[pl.ds((s * _NCHUNK + mc) * 384, 384)], eb, sem).wait()

    def _process(eb, nloc):
        for g in range(_CHUNK // 16):
            sl = pl.ds(g * 16, 16)
            sv = eb[sl]
            dv = eb[pl.ds(_CHUNK + g * 16, 16)]
            wv = eb[pl.ds(2 * _CHUNK + g * 16, 16)]
            inr = (dv >= cbase) & (dv < cbase + _HALF)
            sadj = jnp.where(sv >= _HALF, sv + _PAD, sv)
            plsc.store_compressed(st_s.at[pl.ds(nloc, 16)], sadj, mask=inr)
            plsc.store_compressed(st_d.at[pl.ds(nloc, 16)], dv - cbase,
                                  mask=inr)
            plsc.store_compressed(st_w.at[pl.ds(nloc, 16)], wv, mask=inr)
            nloc = nloc + plsc.all_reduce_population_count(inr)[0]

            @pl.when(nloc >= _CHUNK)
            def _():
                _flush()
                st_s[pl.ds(0, 16)] = st_s[pl.ds(_CHUNK, 16)]
                st_d[pl.ds(0, 16)] = st_d[pl.ds(_CHUNK, 16)]
                st_w[pl.ds(0, 16)] = st_w[pl.ds(_CHUNK, 16)]

            nloc = jnp.where(nloc >= _CHUNK, nloc - _CHUNK, nloc)
        return nloc

    _eload(0, eb0, sem0)

    def _chunk_body(k, nloc):
        _eload(2 * k + 1, eb1, sem1)
        _ewait(2 * k, eb0, sem0)
        nloc = _process(eb0, nloc)

        @pl.when(k < _NCHUNK // 2 - 1)
        def _():
            _eload(2 * k + 2, eb0, sem0)

        _ewait(2 * k + 1, eb1, sem1)
        nloc = _process(eb1, nloc)
        return nloc

    nloc = lax.fori_loop(0, _NCHUNK // 2, _chunk_body, jnp.int32(0))

    for g in range(_CHUNK // 16):
        sl = pl.ds(g * 16, 16)
        keep = (g * 16 + iota) < nloc
        st_s[sl] = jnp.where(keep, st_s[sl], 0)
        st_d[sl] = jnp.where(keep, st_d[sl], _HALF + (g % 4) * 16 + iota)
        st_w[sl] = jnp.where(keep, st_w[sl], 0)
    _flush()

    plsc.subcore_barrier()

    @pl.when(s == 0)
    def _():
        total = plsc.fetch_and_add(cnt.at[0], 0, subcore_id=0)
        cbuf[pl.ds(0, 16)] = jnp.full((16,), total, jnp.int32)
        pltpu.sync_copy(cbuf, counts_hbm.at[pl.ds(c * 16, 16)])


@functools.partial(
    pl.kernel,
    mesh=_mesh(),
    out_type=jax.ShapeDtypeStruct((_NP, _D), jnp.float32),
    scratch_types=[
        pltpu.VMEM_SHARED((_HP, _D), jnp.float32),
        pltpu.VMEM((3 * _CHUNK,), jnp.int32),
        pltpu.VMEM((3 * _CHUNK,), jnp.int32),
        pltpu.VMEM((_CHUNK,), jnp.int32),
        pltpu.VMEM((_CHUNK,), jnp.int32),
        pltpu.VMEM((_CHUNK,), jnp.int32),
        pltpu.VMEM((_CHUNK,), jnp.int32),
        pltpu.VMEM((_CHUNK, _D), jnp.float32),
        pltpu.VMEM((_CHUNK, _D), jnp.float32),
        pltpu.VMEM((_ZROWS, _D), jnp.float32),
        pltpu.VMEM((16,), jnp.int32),
        pltpu.SemaphoreType.DMA,
        pltpu.SemaphoreType.DMA,
    ],
    compiler_params=_CPARAMS,
)
def _layer(part_hbm, counts_hbm, emb_hbm, out_hbm,
           acc, eb0, eb1, sv0, sv1, dv0, dv1,
           rows0, rows1, zv, cbuf, sem0, sem1):
    c = lax.axis_index("c")
    s = lax.axis_index("s")
    obase = c * _REGW

    def _zero_body(i, _):
        r = i // 4
        j = i % 4
        zv[r, pl.ds(j * 16, 16)] = jnp.zeros((16,), jnp.float32)
        return 0

    lax.fori_loop(0, _ZROWS * 4, _zero_body, 0, unroll=4)

    def _zinit(k, _):
        pltpu.sync_copy(zv, acc.at[pl.ds(s * _TROWS + k * _ZROWS, _ZROWS)])
        return 0

    lax.fori_loop(0, _TROWS // _ZROWS, _zinit, 0)

    pltpu.sync_copy(counts_hbm.at[pl.ds(c * 16, 16)], cbuf)
    nch = cbuf[pl.ds(0, 16)][0]
    tc = (nch + 15 - s) // 16
    plsc.subcore_barrier()

    def _issue(i, eb, sv, dv, rows, sem):
        pltpu.sync_copy(
            part_hbm.at[pl.ds(obase + (s + 16 * i) * 384, 384)], eb)
        for g in range(_CHUNK // 16):
            sl = pl.ds(g * 16, 16)
            sv[sl] = eb[sl]
            dv[sl] = eb[pl.ds(_CHUNK + g * 16, 16)]
        pltpu.async_copy(emb_hbm.at[sv], rows, sem)

    def _finish(eb, sv, dv, rows, sem):
        pltpu.make_async_copy(emb_hbm.at[sv], rows, sem).wait()

        def _scale_body(g, _):
            wg = plsc.bitcast(eb[pl.ds(2 * _CHUNK + g * 16, 16)], jnp.float32)
            rbase = g * 16
            for i in range(16):
                wr = wg[i]
                for j in range(4):
                    sl2 = pl.ds(j * 16, 16)
                    rows[rbase + i, sl2] = rows[rbase + i, sl2] * wr
            return 0

        lax.fori_loop(0, _CHUNK // 16, _scale_body, 0)
        pltpu.sync_copy(rows, acc.at[dv], add=True)

    @pl.when(tc > 0)
    def _():
        _issue(0, eb0, sv0, dv0, rows0, sem0)

    def _pipe_body(k, _):
        @pl.when(2 * k + 1 < tc)
        def _():
            _issue(2 * k + 1, eb1, sv1, dv1, rows1, sem1)

        _finish(eb0, sv0, dv0, rows0, sem0)

        @pl.when(2 * k + 2 < tc)
        def _():
            _issue(2 * k + 2, eb0, sv0, dv0, rows0, sem0)

        @pl.when(2 * k + 1 < tc)
        def _():
            _finish(eb1, sv1, dv1, rows1, sem1)

        return 0

    lax.fori_loop(0, (tc + 1) // 2, _pipe_body, 0)
    plsc.subcore_barrier()

    def _wback(k, _):
        off = s * _TROWS + k * _ZROWS
        pltpu.sync_copy(acc.at[pl.ds(off, _ZROWS)],
                        out_hbm.at[pl.ds(c * _HP + off, _ZROWS)])
        return 0

    lax.fori_loop(0, _TROWS // _ZROWS, _wback, 0)


def kernel(edge_index, edge_weight, user_table, item_table):
    src = edge_index[1].astype(jnp.int32)
    dst = edge_index[0].astype(jnp.int32)
    w32 = lax.bitcast_convert_type(edge_weight.astype(jnp.float32), jnp.int32)
    pad = _EPAD - _E
    src = jnp.concatenate([src, jnp.zeros((pad,), jnp.int32)])
    dst = jnp.concatenate([dst, jnp.zeros((pad,), jnp.int32)])
    w32 = jnp.concatenate([w32, jnp.zeros((pad,), jnp.int32)])
    e = jnp.stack([src.reshape(-1, _CHUNK), dst.reshape(-1, _CHUNK),
                   w32.reshape(-1, _CHUNK)], axis=1).reshape(-1)

    part, counts = _partition(e)

    z8 = jnp.zeros((_PAD, _D), jnp.float32)
    emb = jnp.concatenate([user_table, z8, item_table, z8], axis=0)

    total = emb
    cur = emb
    for _ in range(_LAYERS):
        cur = _layer(part, counts, cur)
        total = total + cur
    mean = total * (1.0 / (_LAYERS + 1))
    return (mean[:_NUM_USER], mean[_HP:_HP + _NUM_ITEM])
